# Initial kernel scaffold; baseline (speedup 1.0000x reference)
import jax
import jax.numpy as jnp
from jax.experimental import pallas as pl


def kernel(x, lengths, table, W1, b1, W2, b2):
    raise NotImplementedError("write your pallas kernel here")


# speedup vs baseline: 12.9345x; 12.9345x over previous
import functools

import jax
import jax.numpy as jnp
from jax import lax
from jax.experimental import pallas as pl
from jax.experimental.pallas import tpu as pltpu
from jax.experimental.pallas import tpu_sc as plsc

NC, NS, LANES = 2, 16, 16
NW = NC * NS

KA, KB = 104, 96


def _sc_pooled_sums(x, table):
    B, L = x.shape
    V, D = table.shape
    x = x.reshape(-1)
    rows_w = B // NW
    nchunk = D // LANES

    mesh = plsc.VectorSubcoreMesh(core_axis_name="c", subcore_axis_name="s")

    def body(x_hbm, table_hbm, out_hbm, idx_v, buf0, buf1, out_v, sem0, sem1):
        wid = lax.axis_index("s") * NC + lax.axis_index("c")
        base = wid * rows_w
        pltpu.sync_copy(x_hbm.at[pl.ds(base * L, rows_w * L)], idx_v)

        bufs = (buf0, buf1)
        sems = (sem0, sem1)

        def start_row(b, buf, sem):
            pltpu.make_async_copy(
                table_hbm.at[idx_v.at[pl.ds(b * L, KA)]],
                buf.at[pl.ds(0, KA)], sem).start()
            pltpu.make_async_copy(
                table_hbm.at[idx_v.at[pl.ds(b * L + KA, KB)]],
                buf.at[pl.ds(KA, KB)], sem).start()

        def wait_row(b, buf, sem):
            pltpu.make_async_copy(
                table_hbm.at[idx_v.at[pl.ds(b * L, KA)]],
                buf.at[pl.ds(0, KA)], sem).wait()
            pltpu.make_async_copy(
                table_hbm.at[idx_v.at[pl.ds(b * L + KA, KB)]],
                buf.at[pl.ds(KA, KB)], sem).wait()

        def reduce_row(b, buf):
            def rbody(r, acc):
                a0 = tuple(acc[j] + buf[2 * r, pl.ds(LANES * j, LANES)]
                           for j in range(nchunk))
                return tuple(a0[j] + buf[2 * r + 1, pl.ds(LANES * j, LANES)]
                             for j in range(nchunk))

            acc = lax.fori_loop(
                0, L // 2, rbody,
                tuple(jnp.zeros((LANES,), jnp.float32) for _ in range(nchunk)))
            for j in range(nchunk):
                out_v[b, pl.ds(LANES * j, LANES)] = acc[j]

        start_row(0, bufs[0], sems[0])

        def pair(i, carry):
            b0 = 2 * i
            start_row(b0 + 1, bufs[1], sems[1])
            wait_row(b0, bufs[0], sems[0])
            reduce_row(b0, bufs[0])

            @pl.when(b0 + 2 < rows_w)
            def _():
                start_row(b0 + 2, bufs[0], sems[0])

            wait_row(b0 + 1, bufs[1], sems[1])
            reduce_row(b0 + 1, bufs[1])
            return carry

        lax.fori_loop(0, rows_w // 2, pair, 0)
        pltpu.sync_copy(out_v, out_hbm.at[pl.ds(base, rows_w)])

    return pl.kernel(
        body,
        out_type=jax.ShapeDtypeStruct((B, D), jnp.float32),
        mesh=mesh,
        scratch_types=[
            pltpu.VMEM((rows_w * L,), jnp.int32),
            pltpu.VMEM((L, D), jnp.float32),
            pltpu.VMEM((L, D), jnp.float32),
            pltpu.VMEM((rows_w, D), jnp.float32),
            pltpu.SemaphoreType.DMA,
            pltpu.SemaphoreType.DMA,
        ],
    )(x, table)


def _tc_mlp(sums, inv_len, W1, b1, W2, b2):
    B, D = sums.shape
    H = W1.shape[1]
    C = W2.shape[1]
    BLK = 512

    def body(s_ref, il_ref, w1_ref, b1_ref, w2_ref, b2_ref, o_ref):
        rep = s_ref[...] * il_ref[...]
        h = jnp.dot(rep, w1_ref[...], preferred_element_type=jnp.float32)
        h = jnp.maximum(h + b1_ref[...], 0.0)
        o_ref[...] = (jnp.dot(h, w2_ref[...], preferred_element_type=jnp.float32)
                      + b2_ref[...])

    grid = (B // BLK,)
    return pl.pallas_call(
        body,
        grid=grid,
        in_specs=[
            pl.BlockSpec((BLK, D), lambda i: (i, 0)),
            pl.BlockSpec((BLK, 1), lambda i: (i, 0)),
            pl.BlockSpec((D, H), lambda i: (0, 0)),
            pl.BlockSpec((1, H), lambda i: (0, 0)),
            pl.BlockSpec((H, C), lambda i: (0, 0)),
            pl.BlockSpec((1, C), lambda i: (0, 0)),
        ],
        out_specs=pl.BlockSpec((BLK, C), lambda i: (i, 0)),
        out_shape=jax.ShapeDtypeStruct((B, C), jnp.float32),
    )(sums, inv_len, W1, b1, W2, b2)


@jax.jit
def kernel(x, lengths, table, W1, b1, W2, b2):
    sums = _sc_pooled_sums(x, table)
    inv_len = (1.0 / lengths.astype(jnp.float32)).reshape(-1, 1)
    return _tc_mlp(sums, inv_len, W1, b1.reshape(1, -1), W2, b2.reshape(1, -1))
